# Initial kernel scaffold; baseline (speedup 1.0000x reference)
import jax
import jax.numpy as jnp
from jax.experimental import pallas as pl


def kernel(x, random_projection, codebook):
    raise NotImplementedError("write your pallas kernel here")


# speedup vs baseline: 1.5383x; 1.5383x over previous
import functools

import jax
import jax.numpy as jnp
from jax.experimental import pallas as pl


_EPS = 1e-12
_BIG = 2**30


def _rpq_body(x_ref, rp_ref, cbt_ref, out_ref):
    proj = jnp.dot(x_ref[...], rp_ref[...], preferred_element_type=jnp.float32)
    norm = jnp.sqrt(jnp.sum(proj * proj, axis=1, keepdims=True))
    projn = proj / jnp.maximum(norm, _EPS)

    cbt = cbt_ref[...]
    csq = jnp.sum(cbt * cbt, axis=0, keepdims=True)
    inv = 1.0 / jnp.maximum(jnp.sqrt(csq), _EPS)
    cbn = cbt * inv

    scores = jnp.dot(projn, cbn, preferred_element_type=jnp.float32)
    cbnsq = csq * inv * inv
    scores = scores - 0.5 * cbnsq

    m = jnp.max(scores, axis=1, keepdims=True)
    iota = jax.lax.broadcasted_iota(jnp.int32, scores.shape, 1)
    idx = jnp.min(jnp.where(scores == m, iota, _BIG), axis=1)
    out_ref[0, 0, :] = idx.astype(jnp.int32)


@functools.partial(jax.jit, static_argnames=())
def _rpq(x2, rp, cbt):
    bn, d = x2.shape
    k = cbt.shape[1]
    block_rows = 512
    nb = bn // block_rows
    out = pl.pallas_call(
        _rpq_body,
        grid=(nb,),
        in_specs=[
            pl.BlockSpec((block_rows, d), lambda i: (i, 0)),
            pl.BlockSpec((d, rp.shape[1]), lambda i: (0, 0)),
            pl.BlockSpec((cbt.shape[0], k), lambda i: (0, 0)),
        ],
        out_specs=pl.BlockSpec((1, 1, block_rows), lambda i: (i, 0, 0)),
        out_shape=jax.ShapeDtypeStruct((nb, 1, block_rows), jnp.int32),
    )(x2, rp, cbt)
    return out.reshape(bn)


def kernel(x, random_projection, codebook):
    b, n, d = x.shape
    x2 = x.reshape(b * n, d)
    cbt = codebook.T
    idx = _rpq(x2, random_projection, cbt)
    return idx.reshape(b, n)
